# Initial kernel scaffold; baseline (speedup 1.0000x reference)
import jax
import jax.numpy as jnp
from jax.experimental import pallas as pl


def kernel(x, edge_index, edge_attr, batch, params):
    raise NotImplementedError("write your pallas kernel here")


# speedup vs baseline: 7.5934x; 7.5934x over previous
import functools

import jax
import jax.numpy as jnp
from jax import lax
from jax.experimental import pallas as pl
from jax.experimental.pallas import tpu as pltpu
from jax.experimental.pallas import tpu_sc as plsc

N = 10000
E = 320000
H = 128
ED = 16
OUT = 10
G = 64

NPAD = 10240
NUM_WORKERS = 32


def _dg(v, idx):
    dn = lax.GatherDimensionNumbers(offset_dims=(), collapsed_slice_dims=(0,),
                                    start_index_map=(0,))
    return lax.gather(v, idx[:, None], dn, (1,),
                      mode=lax.GatherScatterMode.PROMISE_IN_BOUNDS)
EDGES_PER_WORKER = E // NUM_WORKERS
CHUNK = 80
NCHUNK = EDGES_PER_WORKER // CHUNK


def _sc_gat_body(xl_hbm, xr_hbm, e_hbm, src_hbm, dst_hbm, att_hbm,
                 out_hbm, den_hbm,
                 srcb, dstb, xlb, xrb, eb, exb, attb, zb, zd, sem,
                 out_sh, den_sh):
    c = lax.axis_index("c")
    s = lax.axis_index("s")
    wid = s * 2 + c

    def _zrow(i, _):
        for j in range(H // 16):
            zb[i, pl.ds(j * 16, 16)] = jnp.zeros((16,), jnp.float32)
        return 0
    lax.fori_loop(0, 16, _zrow, 0)

    def _zd(i, _):
        zd[pl.ds(i * 16, 16)] = jnp.zeros((16,), jnp.float32)
        return 0
    lax.fori_loop(0, 40, _zd, 0)

    def _zcp(i, _):
        pltpu.sync_copy(zb, out_sh.at[pl.ds(s * 640 + i * 16, 16)])
        return 0
    lax.fori_loop(0, 40, _zcp, 0)
    pltpu.sync_copy(zd, den_sh.at[pl.ds(s * 640, 640)])

    pltpu.sync_copy(att_hbm, attb)
    att06 = [attb[pl.ds(16 * j, 16)] * 0.6 for j in range(H // 16)]
    att04 = [attb[pl.ds(16 * j, 16)] * 0.4 for j in range(H // 16)]

    plsc.subcore_barrier()

    def _chunk(k, _):
        base = wid * EDGES_PER_WORKER + k * CHUNK
        pltpu.sync_copy(src_hbm.at[pl.ds(base, CHUNK)], srcb)
        pltpu.sync_copy(dst_hbm.at[pl.ds(base, CHUNK)], dstb)
        pltpu.sync_copy(e_hbm.at[pl.ds(base, CHUNK)], eb)
        pltpu.async_copy(xl_hbm.at[srcb], xlb, sem).wait()
        pltpu.async_copy(xr_hbm.at[dstb], xrb, sem).wait()

        riota = lax.broadcasted_iota(jnp.int32, (16,), 0)

        def _group(gi, _):
            eg = gi * 16

            def _edge(i, tot):
                acc = jnp.zeros((16,), jnp.float32)
                for j in range(H // 16):
                    sl = pl.ds(j * 16, 16)
                    v = xlb[eg + i, sl] + xrb[eg + i, sl] + eb[eg + i, sl]
                    acc = acc + att06[j] * v + att04[j] * jnp.abs(v)
                for sh in (8, 4, 2, 1):
                    acc = acc + _dg(acc, riota ^ sh)
                return jnp.where(riota == i, acc, tot)

            tot = lax.fori_loop(0, 16, _edge, jnp.zeros((16,), jnp.float32))
            exv = jnp.exp(tot)
            exb[pl.ds(eg, 16)] = exv

            def _wt(i, _):
                bv = _dg(exv, riota * 0 + i)
                for j in range(H // 16):
                    sl = pl.ds(j * 16, 16)
                    xlb[eg + i, sl] = xlb[eg + i, sl] * bv
                return 0

            lax.fori_loop(0, 16, _wt, 0)
            return 0
        lax.fori_loop(0, CHUNK // 16, _group, 0)

        pltpu.sync_copy(exb, den_sh.at[dstb], add=True)
        pltpu.sync_copy(xlb, out_sh.at[dstb], add=True)
        return 0

    lax.fori_loop(0, NCHUNK, _chunk, 0)

    plsc.subcore_barrier()

    @pl.when(s == 0)
    def _copy_out():
        pltpu.sync_copy(out_sh, out_hbm.at[c])
        pltpu.sync_copy(den_sh, den_hbm.at[c])


def _sc_gat_layer(xl, xr, e, src, dst, att):
    mesh = plsc.VectorSubcoreMesh(core_axis_name="c", subcore_axis_name="s")

    f = pl.kernel(
        _sc_gat_body,
        out_type=[
            jax.ShapeDtypeStruct((2, NPAD, H), jnp.float32),
            jax.ShapeDtypeStruct((2, NPAD), jnp.float32),
        ],
        mesh=mesh,
        scratch_types=[
            pltpu.VMEM((CHUNK,), jnp.int32),
            pltpu.VMEM((CHUNK,), jnp.int32),
            pltpu.VMEM((CHUNK, H), jnp.float32),
            pltpu.VMEM((CHUNK, H), jnp.float32),
            pltpu.VMEM((CHUNK, H), jnp.float32),
            pltpu.VMEM((CHUNK,), jnp.float32),
            pltpu.VMEM((H,), jnp.float32),
            pltpu.VMEM((16, H), jnp.float32),
            pltpu.VMEM((640,), jnp.float32),
            pltpu.SemaphoreType.DMA,
            pltpu.VMEM_SHARED((NPAD, H), jnp.float32),
            pltpu.VMEM_SHARED((NPAD,), jnp.float32),
        ],
    )
    acc, den = f(xl, xr, e, src, dst, att)
    return acc[:, :N, :], den[:, :N]


def _e_proj_body(ea_ref, w_ref, o1, o2, o3, o4):
    ea = ea_ref[...]
    w = w_ref[...]
    for i, o in enumerate((o1, o2, o3, o4)):
        o[...] = jnp.dot(ea, w[i], preferred_element_type=jnp.float32)


def _e_proj(edge_attr, w_stack):
    BE = 4000
    grid = (E // BE,)
    return pl.pallas_call(
        _e_proj_body,
        grid=grid,
        in_specs=[
            pl.BlockSpec((BE, ED), lambda i: (i, 0)),
            pl.BlockSpec((4, ED, H), lambda i: (0, 0, 0)),
        ],
        out_specs=[pl.BlockSpec((BE, H), lambda i: (i, 0))] * 4,
        out_shape=[jax.ShapeDtypeStruct((E, H), jnp.float32)] * 4,
    )(edge_attr, w_stack)


def _lin1_body(x_ref, wl_ref, bl_ref, wr_ref, br_ref, xl_ref, xr_ref):
    x = x_ref[...]
    xl_ref[...] = jnp.dot(x, wl_ref[...], preferred_element_type=jnp.float32) + bl_ref[...]
    xr_ref[...] = jnp.dot(x, wr_ref[...], preferred_element_type=jnp.float32) + br_ref[...]


def _lin1(x, wl, bl, wr, br):
    return pl.pallas_call(
        _lin1_body,
        out_shape=[jax.ShapeDtypeStruct((N, H), jnp.float32)] * 2,
    )(x, wl, bl.reshape(1, H), wr, br.reshape(1, H))


def _post_gat(acc_ref, den_ref, bias_ref, g_ref, be_ref, a_ref):
    acc = acc_ref[0] + acc_ref[1]
    den = den_ref[0] + den_ref[1]
    h = acc / (den + 1e-16) + bias_ref[...]
    mu = jnp.mean(h, axis=0, keepdims=True)
    var = jnp.mean((h - mu) ** 2, axis=0, keepdims=True)
    hn = g_ref[...] * (h - mu) / jnp.sqrt(var + 1e-5) + be_ref[...]
    a = a_ref[0, 0]
    return jnp.where(hn >= 0, hn, a * hn)


def _mid_body(nprev, refs):
    (acc_ref, den_ref, bias_ref, g_ref, be_ref, a_ref) = refs[:6]
    prev = refs[6:6 + nprev]
    wl_ref, bl_ref, wr_ref, br_ref = refs[6 + nprev:6 + nprev + 4]
    h_ref, xl_ref, xr_ref = refs[6 + nprev + 4:]
    hv = _post_gat(acc_ref, den_ref, bias_ref, g_ref, be_ref, a_ref)
    h_ref[...] = hv
    inp = jnp.concatenate([p[...] for p in prev] + [hv], axis=1)
    xl_ref[...] = jnp.dot(inp, wl_ref[...], preferred_element_type=jnp.float32) + bl_ref[...]
    xr_ref[...] = jnp.dot(inp, wr_ref[...], preferred_element_type=jnp.float32) + br_ref[...]


def _mid(acc, den, bias, g, be, a, prev_hs, wl, bl, wr, br):
    nprev = len(prev_hs)

    def body(*refs):
        _mid_body(nprev, refs)

    return pl.pallas_call(
        body,
        out_shape=[jax.ShapeDtypeStruct((N, H), jnp.float32)] * 3,
    )(acc, den.reshape(2, N, 1), bias.reshape(1, H), g.reshape(1, H),
      be.reshape(1, H), a.reshape(1, 1), *prev_hs,
      wl, bl.reshape(1, H), wr, br.reshape(1, H))


def _final_body(acc_ref, den_ref, bias_ref, h1_ref, h2_ref, h3_ref, b_ref,
                wm1_ref, bm1_ref, gm_ref, bem_ref, am_ref, wm2_ref, bm2_ref,
                lg_ref, pr_ref, pd_ref):
    acc = acc_ref[0] + acc_ref[1]
    den = den_ref[0] + den_ref[1]
    h4 = acc / (den + 1e-16) + bias_ref[...]
    hcat = jnp.concatenate([h1_ref[...], h2_ref[...], h3_ref[...], h4], axis=1)
    giota = lax.broadcasted_iota(jnp.int32, (1, G), 1)
    oh = (b_ref[...] == giota).astype(jnp.float32)
    pooled = lax.dot_general(oh, hcat, (((0,), (0,)), ((), ())),
                             preferred_element_type=jnp.float32)
    h = jnp.dot(pooled, wm1_ref[...], preferred_element_type=jnp.float32) + bm1_ref[...]
    mu = jnp.mean(h, axis=0, keepdims=True)
    var = jnp.mean((h - mu) ** 2, axis=0, keepdims=True)
    hn = gm_ref[...] * (h - mu) / jnp.sqrt(var + 1e-5) + bem_ref[...]
    am = am_ref[0, 0]
    h = jnp.where(hn >= 0, hn, am * hn)
    logits = jnp.dot(h, wm2_ref[...], preferred_element_type=jnp.float32) + bm2_ref[...]
    lg_ref[...] = logits
    ex = jnp.exp(logits - jnp.max(logits, axis=1, keepdims=True))
    prob = ex / jnp.sum(ex, axis=1, keepdims=True)
    pr_ref[...] = prob
    pmax = jnp.max(prob, axis=1, keepdims=True)
    cidx = lax.broadcasted_iota(jnp.int32, (G, OUT), 1)
    pd_ref[...] = jnp.min(jnp.where(prob == pmax, cidx, OUT), axis=1, keepdims=True)


def _final(acc, den, bias, h1, h2, h3, batch, wm1, bm1, gm, bem, am, wm2, bm2):
    return pl.pallas_call(
        _final_body,
        out_shape=[
            jax.ShapeDtypeStruct((G, OUT), jnp.float32),
            jax.ShapeDtypeStruct((G, OUT), jnp.float32),
            jax.ShapeDtypeStruct((G, 1), jnp.int32),
        ],
    )(acc, den.reshape(2, N, 1), bias.reshape(1, H), h1, h2, h3,
      batch.reshape(N, 1), wm1, bm1.reshape(1, H), gm.reshape(1, H),
      bem.reshape(1, H), am.reshape(1, 1), wm2, bm2.reshape(1, OUT))


def kernel(x, edge_index, edge_attr, batch, params):
    p = params
    src = edge_index[0]
    dst = edge_index[1]

    w_stack = jnp.stack([p['We1'], p['We2'], p['We3'], p['We4']])
    e1, e2, e3, e4 = _e_proj(edge_attr, w_stack)

    xl, xr = _lin1(x, p['Wl1'], p['bl1'], p['Wr1'], p['br1'])
    acc1, den1 = _sc_gat_layer(xl, xr, e1, src, dst, p['att1'])

    h1, xl, xr = _mid(acc1, den1, p['bias1'], p['g1'], p['be1'], p['a1'],
                      [], p['Wl2'], p['bl2'], p['Wr2'], p['br2'])
    acc2, den2 = _sc_gat_layer(xl, xr, e2, src, dst, p['att2'])

    h2, xl, xr = _mid(acc2, den2, p['bias2'], p['g2'], p['be2'], p['a2'],
                      [h1], p['Wl3'], p['bl3'], p['Wr3'], p['br3'])
    acc3, den3 = _sc_gat_layer(xl, xr, e3, src, dst, p['att3'])

    h3, xl, xr = _mid(acc3, den3, p['bias3'], p['g3'], p['be3'], p['a3'],
                      [h1, h2], p['Wl4'], p['bl4'], p['Wr4'], p['br4'])
    acc4, den4 = _sc_gat_layer(xl, xr, e4, src, dst, p['att4'])

    logits, prob, pred = _final(acc4, den4, p['bias4'], h1, h2, h3, batch,
                                p['Wm1'], p['bm1'], p['gm'], p['bem'],
                                p['am'], p['Wm2'], p['bm2'])
    return logits, prob, pred.reshape(G)


# speedup vs baseline: 7.7828x; 1.0249x over previous
import functools

import jax
import jax.numpy as jnp
from jax import lax
from jax.experimental import pallas as pl
from jax.experimental.pallas import tpu as pltpu
from jax.experimental.pallas import tpu_sc as plsc

N = 10000
E = 320000
H = 128
ED = 16
OUT = 10
G = 64

NPAD = 10240
NUM_WORKERS = 32


def _dg(v, idx):
    dn = lax.GatherDimensionNumbers(offset_dims=(), collapsed_slice_dims=(0,),
                                    start_index_map=(0,))
    return lax.gather(v, idx[:, None], dn, (1,),
                      mode=lax.GatherScatterMode.PROMISE_IN_BOUNDS)
EDGES_PER_WORKER = E // NUM_WORKERS
CHUNK = 16
NCHUNK = EDGES_PER_WORKER // CHUNK


def _sc_gat_body(xl_hbm, xr_hbm, e_hbm, src_hbm, dst_hbm, att_hbm,
                 out_hbm, den_hbm,
                 srcb0, dstb0, eb0, xlb0, xrb0, exb0,
                 srcb1, dstb1, eb1, xlb1, xrb1, exb1,
                 attb, zb, zd, lsem0, gsem0, lsem1, gsem1,
                 out_sh, den_sh):
    slot0 = (srcb0, dstb0, eb0, xlb0, xrb0, exb0, lsem0, gsem0)
    slot1 = (srcb1, dstb1, eb1, xlb1, xrb1, exb1, lsem1, gsem1)
    c = lax.axis_index("c")
    s = lax.axis_index("s")
    wid = s * 2 + c

    def _zrow(i, _):
        for j in range(H // 16):
            zb[i, pl.ds(j * 16, 16)] = jnp.zeros((16,), jnp.float32)
        return 0
    lax.fori_loop(0, 16, _zrow, 0)

    def _zd(i, _):
        zd[pl.ds(i * 16, 16)] = jnp.zeros((16,), jnp.float32)
        return 0
    lax.fori_loop(0, 40, _zd, 0)

    def _zcp(i, _):
        pltpu.sync_copy(zb, out_sh.at[pl.ds(s * 640 + i * 16, 16)])
        return 0
    lax.fori_loop(0, 40, _zcp, 0)
    pltpu.sync_copy(zd, den_sh.at[pl.ds(s * 640, 640)])

    pltpu.sync_copy(att_hbm, attb)
    att06 = [attb[pl.ds(16 * j, 16)] * 0.6 for j in range(H // 16)]
    att04 = [attb[pl.ds(16 * j, 16)] * 0.4 for j in range(H // 16)]

    plsc.subcore_barrier()

    riota = lax.broadcasted_iota(jnp.int32, (16,), 0)

    def _fire_linear(i, slot):
        srcb, dstb, eb, xlb, xrb, exb, lsem, gsem = slot
        base = wid * EDGES_PER_WORKER + i * CHUNK
        pltpu.async_copy(src_hbm.at[pl.ds(base, CHUNK)], srcb, lsem)
        pltpu.async_copy(dst_hbm.at[pl.ds(base, CHUNK)], dstb, lsem)
        pltpu.async_copy(e_hbm.at[pl.ds(base, CHUNK)], eb, lsem)

    def _wait_linear(slot):
        srcb, dstb, eb, xlb, xrb, exb, lsem, gsem = slot
        pltpu.make_async_copy(src_hbm.at[pl.ds(0, CHUNK)], srcb, lsem).wait()
        pltpu.make_async_copy(dst_hbm.at[pl.ds(0, CHUNK)], dstb, lsem).wait()
        pltpu.make_async_copy(e_hbm.at[pl.ds(0, CHUNK)], eb, lsem).wait()

    def _fire_gather(slot):
        srcb, dstb, eb, xlb, xrb, exb, lsem, gsem = slot
        pltpu.async_copy(xl_hbm.at[srcb], xlb, gsem)
        pltpu.async_copy(xr_hbm.at[dstb], xrb, gsem)

    def _wait_gather(slot):
        srcb, dstb, eb, xlb, xrb, exb, lsem, gsem = slot
        pltpu.make_async_copy(xl_hbm.at[srcb], xlb, gsem).wait()
        pltpu.make_async_copy(xr_hbm.at[dstb], xrb, gsem).wait()

    def _compute(slot):
        srcb, dstb, eb, xlb, xrb, exb, lsem, gsem = slot

        def _edge(i, tot):
            acc = jnp.zeros((16,), jnp.float32)
            for j in range(H // 16):
                sl = pl.ds(j * 16, 16)
                v = xlb[i, sl] + xrb[i, sl] + eb[i, sl]
                acc = acc + att06[j] * v + att04[j] * jnp.abs(v)
            for sh in (8, 4, 2, 1):
                acc = acc + _dg(acc, riota ^ sh)
            return jnp.where(riota == i, acc, tot)

        tot = lax.fori_loop(0, 16, _edge, jnp.zeros((16,), jnp.float32))
        exv = jnp.exp(tot)
        exb[...] = exv

        def _wt(i, _):
            bv = _dg(exv, riota * 0 + i)
            for j in range(H // 16):
                sl = pl.ds(j * 16, 16)
                xlb[i, sl] = xlb[i, sl] * bv
            return 0

        lax.fori_loop(0, 16, _wt, 0)

    def _scatter(slot):
        srcb, dstb, eb, xlb, xrb, exb, lsem, gsem = slot
        pltpu.sync_copy(exb, den_sh.at[dstb], add=True)
        pltpu.sync_copy(xlb, out_sh.at[dstb], add=True)

    def _half(i, cur, nxt):
        _fire_linear(i + 1, nxt)
        _wait_gather(cur)
        _wait_linear(nxt)
        _fire_gather(nxt)
        _compute(cur)
        _scatter(cur)

    _fire_linear(0, slot0)
    _wait_linear(slot0)
    _fire_gather(slot0)

    def _pair(pi, _):
        i = 2 * pi
        _half(i, slot0, slot1)
        _half(i + 1, slot1, slot0)
        return 0
    lax.fori_loop(0, (NCHUNK - 1) // 2, _pair, 0)

    _wait_gather(slot0)
    _compute(slot0)
    _scatter(slot0)

    plsc.subcore_barrier()

    @pl.when(s == 0)
    def _copy_out():
        pltpu.sync_copy(out_sh, out_hbm.at[c])
        pltpu.sync_copy(den_sh, den_hbm.at[c])


def _sc_gat_layer(xl, xr, e, src, dst, att):
    mesh = plsc.VectorSubcoreMesh(core_axis_name="c", subcore_axis_name="s")

    f = pl.kernel(
        _sc_gat_body,
        out_type=[
            jax.ShapeDtypeStruct((2, NPAD, H), jnp.float32),
            jax.ShapeDtypeStruct((2, NPAD), jnp.float32),
        ],
        mesh=mesh,
        scratch_types=(
            [
                pltpu.VMEM((CHUNK,), jnp.int32),
                pltpu.VMEM((CHUNK,), jnp.int32),
                pltpu.VMEM((CHUNK, H), jnp.float32),
                pltpu.VMEM((CHUNK, H), jnp.float32),
                pltpu.VMEM((CHUNK, H), jnp.float32),
                pltpu.VMEM((CHUNK,), jnp.float32),
            ] * 2
            + [
                pltpu.VMEM((H,), jnp.float32),
                pltpu.VMEM((16, H), jnp.float32),
                pltpu.VMEM((640,), jnp.float32),
                pltpu.SemaphoreType.DMA,
                pltpu.SemaphoreType.DMA,
                pltpu.SemaphoreType.DMA,
                pltpu.SemaphoreType.DMA,
                pltpu.VMEM_SHARED((NPAD, H), jnp.float32),
                pltpu.VMEM_SHARED((NPAD,), jnp.float32),
            ]
        ),
    )
    acc, den = f(xl, xr, e, src, dst, att)
    return acc[:, :N, :], den[:, :N]


def _e_proj_body(ea_ref, w_ref, o1, o2, o3, o4):
    ea = ea_ref[...]
    w = w_ref[...]
    for i, o in enumerate((o1, o2, o3, o4)):
        o[...] = jnp.dot(ea, w[i], preferred_element_type=jnp.float32)


def _e_proj(edge_attr, w_stack):
    BE = 4000
    grid = (E // BE,)
    return pl.pallas_call(
        _e_proj_body,
        grid=grid,
        in_specs=[
            pl.BlockSpec((BE, ED), lambda i: (i, 0)),
            pl.BlockSpec((4, ED, H), lambda i: (0, 0, 0)),
        ],
        out_specs=[pl.BlockSpec((BE, H), lambda i: (i, 0))] * 4,
        out_shape=[jax.ShapeDtypeStruct((E, H), jnp.float32)] * 4,
    )(edge_attr, w_stack)


def _lin1_body(x_ref, wl_ref, bl_ref, wr_ref, br_ref, xl_ref, xr_ref):
    x = x_ref[...]
    xl_ref[...] = jnp.dot(x, wl_ref[...], preferred_element_type=jnp.float32) + bl_ref[...]
    xr_ref[...] = jnp.dot(x, wr_ref[...], preferred_element_type=jnp.float32) + br_ref[...]


def _lin1(x, wl, bl, wr, br):
    return pl.pallas_call(
        _lin1_body,
        out_shape=[jax.ShapeDtypeStruct((N, H), jnp.float32)] * 2,
    )(x, wl, bl.reshape(1, H), wr, br.reshape(1, H))


def _post_gat(acc_ref, den_ref, bias_ref, g_ref, be_ref, a_ref):
    acc = acc_ref[0] + acc_ref[1]
    den = den_ref[0] + den_ref[1]
    h = acc / (den + 1e-16) + bias_ref[...]
    mu = jnp.mean(h, axis=0, keepdims=True)
    var = jnp.mean((h - mu) ** 2, axis=0, keepdims=True)
    hn = g_ref[...] * (h - mu) / jnp.sqrt(var + 1e-5) + be_ref[...]
    a = a_ref[0, 0]
    return jnp.where(hn >= 0, hn, a * hn)


def _mid_body(nprev, refs):
    (acc_ref, den_ref, bias_ref, g_ref, be_ref, a_ref) = refs[:6]
    prev = refs[6:6 + nprev]
    wl_ref, bl_ref, wr_ref, br_ref = refs[6 + nprev:6 + nprev + 4]
    h_ref, xl_ref, xr_ref = refs[6 + nprev + 4:]
    hv = _post_gat(acc_ref, den_ref, bias_ref, g_ref, be_ref, a_ref)
    h_ref[...] = hv
    inp = jnp.concatenate([p[...] for p in prev] + [hv], axis=1)
    xl_ref[...] = jnp.dot(inp, wl_ref[...], preferred_element_type=jnp.float32) + bl_ref[...]
    xr_ref[...] = jnp.dot(inp, wr_ref[...], preferred_element_type=jnp.float32) + br_ref[...]


def _mid(acc, den, bias, g, be, a, prev_hs, wl, bl, wr, br):
    nprev = len(prev_hs)

    def body(*refs):
        _mid_body(nprev, refs)

    return pl.pallas_call(
        body,
        out_shape=[jax.ShapeDtypeStruct((N, H), jnp.float32)] * 3,
    )(acc, den.reshape(2, N, 1), bias.reshape(1, H), g.reshape(1, H),
      be.reshape(1, H), a.reshape(1, 1), *prev_hs,
      wl, bl.reshape(1, H), wr, br.reshape(1, H))


def _final_body(acc_ref, den_ref, bias_ref, h1_ref, h2_ref, h3_ref, b_ref,
                wm1_ref, bm1_ref, gm_ref, bem_ref, am_ref, wm2_ref, bm2_ref,
                lg_ref, pr_ref, pd_ref):
    acc = acc_ref[0] + acc_ref[1]
    den = den_ref[0] + den_ref[1]
    h4 = acc / (den + 1e-16) + bias_ref[...]
    hcat = jnp.concatenate([h1_ref[...], h2_ref[...], h3_ref[...], h4], axis=1)
    giota = lax.broadcasted_iota(jnp.int32, (1, G), 1)
    oh = (b_ref[...] == giota).astype(jnp.float32)
    pooled = lax.dot_general(oh, hcat, (((0,), (0,)), ((), ())),
                             preferred_element_type=jnp.float32)
    h = jnp.dot(pooled, wm1_ref[...], preferred_element_type=jnp.float32) + bm1_ref[...]
    mu = jnp.mean(h, axis=0, keepdims=True)
    var = jnp.mean((h - mu) ** 2, axis=0, keepdims=True)
    hn = gm_ref[...] * (h - mu) / jnp.sqrt(var + 1e-5) + bem_ref[...]
    am = am_ref[0, 0]
    h = jnp.where(hn >= 0, hn, am * hn)
    logits = jnp.dot(h, wm2_ref[...], preferred_element_type=jnp.float32) + bm2_ref[...]
    lg_ref[...] = logits
    ex = jnp.exp(logits - jnp.max(logits, axis=1, keepdims=True))
    prob = ex / jnp.sum(ex, axis=1, keepdims=True)
    pr_ref[...] = prob
    pmax = jnp.max(prob, axis=1, keepdims=True)
    cidx = lax.broadcasted_iota(jnp.int32, (G, OUT), 1)
    pd_ref[...] = jnp.min(jnp.where(prob == pmax, cidx, OUT), axis=1, keepdims=True)


def _final(acc, den, bias, h1, h2, h3, batch, wm1, bm1, gm, bem, am, wm2, bm2):
    return pl.pallas_call(
        _final_body,
        out_shape=[
            jax.ShapeDtypeStruct((G, OUT), jnp.float32),
            jax.ShapeDtypeStruct((G, OUT), jnp.float32),
            jax.ShapeDtypeStruct((G, 1), jnp.int32),
        ],
    )(acc, den.reshape(2, N, 1), bias.reshape(1, H), h1, h2, h3,
      batch.reshape(N, 1), wm1, bm1.reshape(1, H), gm.reshape(1, H),
      bem.reshape(1, H), am.reshape(1, 1), wm2, bm2.reshape(1, OUT))


def kernel(x, edge_index, edge_attr, batch, params):
    p = params
    src = edge_index[0]
    dst = edge_index[1]

    w_stack = jnp.stack([p['We1'], p['We2'], p['We3'], p['We4']])
    e1, e2, e3, e4 = _e_proj(edge_attr, w_stack)

    xl, xr = _lin1(x, p['Wl1'], p['bl1'], p['Wr1'], p['br1'])
    acc1, den1 = _sc_gat_layer(xl, xr, e1, src, dst, p['att1'])

    h1, xl, xr = _mid(acc1, den1, p['bias1'], p['g1'], p['be1'], p['a1'],
                      [], p['Wl2'], p['bl2'], p['Wr2'], p['br2'])
    acc2, den2 = _sc_gat_layer(xl, xr, e2, src, dst, p['att2'])

    h2, xl, xr = _mid(acc2, den2, p['bias2'], p['g2'], p['be2'], p['a2'],
                      [h1], p['Wl3'], p['bl3'], p['Wr3'], p['br3'])
    acc3, den3 = _sc_gat_layer(xl, xr, e3, src, dst, p['att3'])

    h3, xl, xr = _mid(acc3, den3, p['bias3'], p['g3'], p['be3'], p['a3'],
                      [h1, h2], p['Wl4'], p['bl4'], p['Wr4'], p['br4'])
    acc4, den4 = _sc_gat_layer(xl, xr, e4, src, dst, p['att4'])

    logits, prob, pred = _final(acc4, den4, p['bias4'], h1, h2, h3, batch,
                                p['Wm1'], p['bm1'], p['gm'], p['bem'],
                                p['am'], p['Wm2'], p['bm2'])
    return logits, prob, pred.reshape(G)


# speedup vs baseline: 10.2572x; 1.3179x over previous
import functools

import numpy as np

import jax
import jax.numpy as jnp
from jax import lax
from jax.experimental import pallas as pl
from jax.experimental.pallas import tpu as pltpu
from jax.experimental.pallas import tpu_sc as plsc

N = 10000
E = 320000
H = 128
ED = 16
OUT = 10
G = 64

NPAD = 10240
NUM_WORKERS = 32


def _dg(v, idx):
    dn = lax.GatherDimensionNumbers(offset_dims=(), collapsed_slice_dims=(0,),
                                    start_index_map=(0,))
    return lax.gather(v, idx[:, None], dn, (1,),
                      mode=lax.GatherScatterMode.PROMISE_IN_BOUNDS)
EDGES_PER_WORKER = E // NUM_WORKERS
CHUNK = 40
NCHUNK = EDGES_PER_WORKER // CHUNK
GROUPS = ((0, 8), (16, 8), (32, 4))

_ILV = np.empty(H, np.int32)
for _b in range(H // 32):
    for _k in range(16):
        _ILV[32 * _b + _k] = 32 * _b + 2 * _k
        _ILV[32 * _b + 16 + _k] = 32 * _b + 2 * _k + 1


def _pcols(w):
    return w[:, _ILV]


def _prows(w):
    idx = np.concatenate([_ILV + H * m for m in range(w.shape[0] // H)])
    return w[idx, :]


def _pvec(v):
    return v[_ILV]


def _sc_gat_body(xl_hbm, xr_hbm, e_hbm, src_hbm, dst_hbm, att_hbm,
                 out_hbm, den_hbm,
                 srcb0, dstb0, eb0, xlb0, xrb0, exb0,
                 srcb1, dstb1, eb1, xlb1, xrb1, exb1,
                 attb, zb, zd, lsem0, gsem0, lsem1, gsem1,
                 out_sh, den_sh):
    slot0 = (srcb0, dstb0, eb0, xlb0, xrb0, exb0, lsem0, gsem0)
    slot1 = (srcb1, dstb1, eb1, xlb1, xrb1, exb1, lsem1, gsem1)
    c = lax.axis_index("c")
    s = lax.axis_index("s")
    wid = s * 2 + c

    def _zrow(i, _):
        for j in range(H // 16):
            zb[i, pl.ds(j * 16, 16)] = jnp.zeros((16,), jnp.float32)
        return 0
    lax.fori_loop(0, 16, _zrow, 0)

    def _zd(i, _):
        zd[pl.ds(i * 16, 16)] = jnp.zeros((16,), jnp.float32)
        return 0
    lax.fori_loop(0, 40, _zd, 0)

    def _zcp(i, _):
        pltpu.sync_copy(zb, out_sh.at[pl.ds(s * 640 + i * 16, 16)])
        return 0
    lax.fori_loop(0, 40, _zcp, 0)
    pltpu.sync_copy(zd, den_sh.at[pl.ds(s * 640, 640)])

    pltpu.sync_copy(att_hbm, attb)
    att06 = [attb[pl.ds(16 * j, 16)] * 0.6 for j in range(H // 16)]
    att04 = [attb[pl.ds(16 * j, 16)] * 0.4 for j in range(H // 16)]

    plsc.subcore_barrier()

    riota = lax.broadcasted_iota(jnp.int32, (16,), 0)

    def _fire_linear(i, slot):
        srcb, dstb, eb, xlb, xrb, exb, lsem, gsem = slot
        base = wid * EDGES_PER_WORKER + i * CHUNK
        pltpu.async_copy(src_hbm.at[pl.ds(base, CHUNK)], srcb, lsem)
        pltpu.async_copy(dst_hbm.at[pl.ds(base, CHUNK)], dstb, lsem)
        pltpu.async_copy(e_hbm.at[pl.ds(base, CHUNK)], eb, lsem)

    def _wait_linear(slot):
        srcb, dstb, eb, xlb, xrb, exb, lsem, gsem = slot
        pltpu.make_async_copy(src_hbm.at[pl.ds(0, CHUNK)], srcb, lsem).wait()
        pltpu.make_async_copy(dst_hbm.at[pl.ds(0, CHUNK)], dstb, lsem).wait()
        pltpu.make_async_copy(e_hbm.at[pl.ds(0, CHUNK)], eb, lsem).wait()

    def _fire_gather(slot):
        srcb, dstb, eb, xlb, xrb, exb, lsem, gsem = slot
        pltpu.async_copy(xl_hbm.at[srcb], xlb, gsem)
        pltpu.async_copy(xr_hbm.at[dstb], xrb, gsem)

    def _wait_gather(slot):
        srcb, dstb, eb, xlb, xrb, exb, lsem, gsem = slot
        pltpu.make_async_copy(xl_hbm.at[srcb], xlb, gsem).wait()
        pltpu.make_async_copy(xr_hbm.at[dstb], xrb, gsem).wait()

    def _compute(slot):
        srcb, dstb, eb, xlb, xrb, exb, lsem, gsem = slot

        def _logit(i):
            prods = []
            for j in range(H // 16):
                sl = pl.ds(16 * j, 16)
                v = xlb[i, sl] + xrb[i, sl] + eb[i, sl]
                prods.append(att06[j] * v + att04[j] * jnp.abs(v))
            while len(prods) > 1:
                prods = [a + b for a, b in zip(prods[::2], prods[1::2])]
            acc = prods[0]
            for sh in (8, 4, 2, 1):
                acc = acc + _dg(acc, riota ^ sh)
            return acc

        for eg, npair in GROUPS:
            def _edge2(ii, tot, eg=eg):
                i = eg + 2 * ii
                a0 = _logit(i)
                a1 = _logit(i + 1)
                tot = jnp.where(riota == 2 * ii, a0, tot)
                return jnp.where(riota == 2 * ii + 1, a1, tot)

            tot = lax.fori_loop(0, npair, _edge2,
                                jnp.zeros((16,), jnp.float32))
            exv = jnp.exp(tot)
            exb[pl.ds(eg, 16)] = exv

            def _wt(ii, _, eg=eg, exv=exv):
                i = eg + 2 * ii
                bv0 = _dg(exv, riota * 0 + 2 * ii)
                bv1 = _dg(exv, riota * 0 + 2 * ii + 1)
                for j in range(H // 16):
                    sl = pl.ds(j * 16, 16)
                    xlb[i, sl] = xlb[i, sl] * bv0
                    xlb[i + 1, sl] = xlb[i + 1, sl] * bv1
                return 0

            lax.fori_loop(0, npair, _wt, 0)

    def _scatter(slot):
        srcb, dstb, eb, xlb, xrb, exb, lsem, gsem = slot
        pltpu.sync_copy(exb.at[pl.ds(0, CHUNK)], den_sh.at[dstb], add=True)
        pltpu.sync_copy(xlb, out_sh.at[dstb], add=True)

    def _half(i, cur, nxt):
        _fire_linear(i + 1, nxt)
        _wait_gather(cur)
        _wait_linear(nxt)
        _fire_gather(nxt)
        _compute(cur)
        _scatter(cur)

    _fire_linear(0, slot0)
    _wait_linear(slot0)
    _fire_gather(slot0)

    def _pair(pi, _):
        i = 2 * pi
        _half(i, slot0, slot1)
        _half(i + 1, slot1, slot0)
        return 0
    lax.fori_loop(0, (NCHUNK - 2) // 2, _pair, 0)

    _half(NCHUNK - 2, slot0, slot1)
    _wait_gather(slot1)
    _compute(slot1)
    _scatter(slot1)

    plsc.subcore_barrier()

    @pl.when(s == 0)
    def _copy_out():
        pltpu.sync_copy(out_sh, out_hbm.at[c])
        pltpu.sync_copy(den_sh, den_hbm.at[c])


def _pack32(x):
    return lax.bitcast_convert_type(
        x.reshape(x.shape[:-1] + (H // 2, 2)), jnp.int32)


def _sc_gat_layer(xl, xr, e, src, dst, att):
    mesh = plsc.VectorSubcoreMesh(core_axis_name="c", subcore_axis_name="s")

    f = pl.kernel(
        _sc_gat_body,
        out_type=[
            jax.ShapeDtypeStruct((2, NPAD, H), jnp.float32),
            jax.ShapeDtypeStruct((2, NPAD), jnp.float32),
        ],
        mesh=mesh,
        scratch_types=(
            [
                pltpu.VMEM((CHUNK,), jnp.int32),
                pltpu.VMEM((CHUNK,), jnp.int32),
                pltpu.VMEM((CHUNK, H), jnp.float32),
                pltpu.VMEM((CHUNK, H), jnp.float32),
                pltpu.VMEM((CHUNK, H), jnp.float32),
                pltpu.VMEM((CHUNK + 8,), jnp.float32),
            ] * 2
            + [
                pltpu.VMEM((H,), jnp.float32),
                pltpu.VMEM((16, H), jnp.float32),
                pltpu.VMEM((640,), jnp.float32),
                pltpu.SemaphoreType.DMA,
                pltpu.SemaphoreType.DMA,
                pltpu.SemaphoreType.DMA,
                pltpu.SemaphoreType.DMA,
                pltpu.VMEM_SHARED((NPAD, H), jnp.float32),
                pltpu.VMEM_SHARED((NPAD,), jnp.float32),
            ]
        ),
    )
    acc, den = f(xl, xr, e, src, dst, att)
    return acc[:, :N, :], den[:, :N]


def _e_proj_body(ea_ref, w_ref, o1, o2, o3, o4):
    ea = ea_ref[...]
    w = w_ref[...]
    for i, o in enumerate((o1, o2, o3, o4)):
        o[...] = jnp.dot(ea, w[i], preferred_element_type=jnp.float32)


def _e_proj(edge_attr, w_stack):
    BE = 4000
    grid = (E // BE,)
    return pl.pallas_call(
        _e_proj_body,
        grid=grid,
        in_specs=[
            pl.BlockSpec((BE, ED), lambda i: (i, 0)),
            pl.BlockSpec((4, ED, H), lambda i: (0, 0, 0)),
        ],
        out_specs=[pl.BlockSpec((BE, H), lambda i: (i, 0))] * 4,
        out_shape=[jax.ShapeDtypeStruct((E, H), jnp.float32)] * 4,
    )(edge_attr, w_stack)


def _lin1_body(x_ref, wl_ref, bl_ref, wr_ref, br_ref, xl_ref, xr_ref):
    x = x_ref[...]
    xl_ref[...] = jnp.dot(x, wl_ref[...], preferred_element_type=jnp.float32) + bl_ref[...]
    xr_ref[...] = jnp.dot(x, wr_ref[...], preferred_element_type=jnp.float32) + br_ref[...]


def _lin1(x, wl, bl, wr, br):
    return pl.pallas_call(
        _lin1_body,
        out_shape=[jax.ShapeDtypeStruct((N, H), jnp.float32)] * 2,
    )(x, wl, bl.reshape(1, H), wr, br.reshape(1, H))


def _post_gat(acc_ref, den_ref, bias_ref, g_ref, be_ref, a_ref):
    acc = acc_ref[0] + acc_ref[1]
    den = den_ref[0] + den_ref[1]
    h = acc / (den + 1e-16) + bias_ref[...]
    mu = jnp.mean(h, axis=0, keepdims=True)
    var = jnp.mean((h - mu) ** 2, axis=0, keepdims=True)
    hn = g_ref[...] * (h - mu) / jnp.sqrt(var + 1e-5) + be_ref[...]
    a = a_ref[0, 0]
    return jnp.where(hn >= 0, hn, a * hn)


def _mid_body(nprev, refs):
    (acc_ref, den_ref, bias_ref, g_ref, be_ref, a_ref) = refs[:6]
    prev = refs[6:6 + nprev]
    wl_ref, bl_ref, wr_ref, br_ref = refs[6 + nprev:6 + nprev + 4]
    h_ref, xl_ref, xr_ref = refs[6 + nprev + 4:]
    hv = _post_gat(acc_ref, den_ref, bias_ref, g_ref, be_ref, a_ref)
    h_ref[...] = hv
    inp = jnp.concatenate([p[...] for p in prev] + [hv], axis=1)
    xl_ref[...] = jnp.dot(inp, wl_ref[...], preferred_element_type=jnp.float32) + bl_ref[...]
    xr_ref[...] = jnp.dot(inp, wr_ref[...], preferred_element_type=jnp.float32) + br_ref[...]


def _mid(acc, den, bias, g, be, a, prev_hs, wl, bl, wr, br):
    nprev = len(prev_hs)

    def body(*refs):
        _mid_body(nprev, refs)

    return pl.pallas_call(
        body,
        out_shape=[jax.ShapeDtypeStruct((N, H), jnp.float32)] * 3,
    )(acc, den.reshape(2, N, 1), bias.reshape(1, H), g.reshape(1, H),
      be.reshape(1, H), a.reshape(1, 1), *prev_hs,
      wl, bl.reshape(1, H), wr, br.reshape(1, H))


def _final_body(acc_ref, den_ref, bias_ref, h1_ref, h2_ref, h3_ref, b_ref,
                wm1_ref, bm1_ref, gm_ref, bem_ref, am_ref, wm2_ref, bm2_ref,
                lg_ref, pr_ref, pd_ref):
    acc = acc_ref[0] + acc_ref[1]
    den = den_ref[0] + den_ref[1]
    h4 = acc / (den + 1e-16) + bias_ref[...]
    hcat = jnp.concatenate([h1_ref[...], h2_ref[...], h3_ref[...], h4], axis=1)
    giota = lax.broadcasted_iota(jnp.int32, (1, G), 1)
    oh = (b_ref[...] == giota).astype(jnp.float32)
    pooled = lax.dot_general(oh, hcat, (((0,), (0,)), ((), ())),
                             preferred_element_type=jnp.float32)
    h = jnp.dot(pooled, wm1_ref[...], preferred_element_type=jnp.float32) + bm1_ref[...]
    mu = jnp.mean(h, axis=0, keepdims=True)
    var = jnp.mean((h - mu) ** 2, axis=0, keepdims=True)
    hn = gm_ref[...] * (h - mu) / jnp.sqrt(var + 1e-5) + bem_ref[...]
    am = am_ref[0, 0]
    h = jnp.where(hn >= 0, hn, am * hn)
    logits = jnp.dot(h, wm2_ref[...], preferred_element_type=jnp.float32) + bm2_ref[...]
    lg_ref[...] = logits
    ex = jnp.exp(logits - jnp.max(logits, axis=1, keepdims=True))
    prob = ex / jnp.sum(ex, axis=1, keepdims=True)
    pr_ref[...] = prob
    pmax = jnp.max(prob, axis=1, keepdims=True)
    cidx = lax.broadcasted_iota(jnp.int32, (G, OUT), 1)
    pd_ref[...] = jnp.min(jnp.where(prob == pmax, cidx, OUT), axis=1, keepdims=True)


def _final(acc, den, bias, h1, h2, h3, batch, wm1, bm1, gm, bem, am, wm2, bm2):
    return pl.pallas_call(
        _final_body,
        out_shape=[
            jax.ShapeDtypeStruct((G, OUT), jnp.float32),
            jax.ShapeDtypeStruct((G, OUT), jnp.float32),
            jax.ShapeDtypeStruct((G, 1), jnp.int32),
        ],
    )(acc, den.reshape(2, N, 1), bias.reshape(1, H), h1, h2, h3,
      batch.reshape(N, 1), wm1, bm1.reshape(1, H), gm.reshape(1, H),
      bem.reshape(1, H), am.reshape(1, 1), wm2, bm2.reshape(1, OUT))


def kernel(x, edge_index, edge_attr, batch, params):
    p = params
    src = edge_index[0]
    dst = edge_index[1]

    w_stack = jnp.stack([_pcols(p['We1']), _pcols(p['We2']),
                         _pcols(p['We3']), _pcols(p['We4'])])
    e1, e2, e3, e4 = _e_proj(edge_attr, w_stack)

    xl, xr = _lin1(x, _pcols(p['Wl1']), _pvec(p['bl1']),
                   _pcols(p['Wr1']), _pvec(p['br1']))
    acc1, den1 = _sc_gat_layer(xl, xr, e1, src, dst, _pvec(p['att1']))

    h1, xl, xr = _mid(acc1, den1, _pvec(p['bias1']), _pvec(p['g1']),
                      _pvec(p['be1']), p['a1'],
                      [], _pcols(_prows(p['Wl2'])), _pvec(p['bl2']),
                      _pcols(_prows(p['Wr2'])), _pvec(p['br2']))
    acc2, den2 = _sc_gat_layer(xl, xr, e2, src, dst, _pvec(p['att2']))

    h2, xl, xr = _mid(acc2, den2, _pvec(p['bias2']), _pvec(p['g2']),
                      _pvec(p['be2']), p['a2'],
                      [h1], _pcols(_prows(p['Wl3'])), _pvec(p['bl3']),
                      _pcols(_prows(p['Wr3'])), _pvec(p['br3']))
    acc3, den3 = _sc_gat_layer(xl, xr, e3, src, dst, _pvec(p['att3']))

    h3, xl, xr = _mid(acc3, den3, _pvec(p['bias3']), _pvec(p['g3']),
                      _pvec(p['be3']), p['a3'],
                      [h1, h2], _pcols(_prows(p['Wl4'])), _pvec(p['bl4']),
                      _pcols(_prows(p['Wr4'])), _pvec(p['br4']))
    acc4, den4 = _sc_gat_layer(xl, xr, e4, src, dst, _pvec(p['att4']))

    logits, prob, pred = _final(acc4, den4, _pvec(p['bias4']), h1, h2, h3,
                                batch, _prows(p['Wm1']), p['bm1'], p['gm'],
                                p['bem'], p['am'], p['Wm2'], p['bm2'])
    return logits, prob, pred.reshape(G)


# speedup vs baseline: 10.3648x; 1.0105x over previous
import functools

import numpy as np

import jax
import jax.numpy as jnp
from jax import lax
from jax.experimental import pallas as pl
from jax.experimental.pallas import tpu as pltpu
from jax.experimental.pallas import tpu_sc as plsc

N = 10000
E = 320000
H = 128
ED = 16
OUT = 10
G = 64

NPAD = 10240
NUM_WORKERS = 32


def _dg(v, idx):
    dn = lax.GatherDimensionNumbers(offset_dims=(), collapsed_slice_dims=(0,),
                                    start_index_map=(0,))
    return lax.gather(v, idx[:, None], dn, (1,),
                      mode=lax.GatherScatterMode.PROMISE_IN_BOUNDS)
EDGES_PER_WORKER = E // NUM_WORKERS
CHUNK = 40
NCHUNK = EDGES_PER_WORKER // CHUNK
GROUPS = ((0, 8), (16, 8), (32, 4))

_ILV = np.empty(H, np.int32)
for _b in range(H // 32):
    for _k in range(16):
        _ILV[32 * _b + _k] = 32 * _b + 2 * _k
        _ILV[32 * _b + 16 + _k] = 32 * _b + 2 * _k + 1


def _pcols(w):
    return w[:, _ILV]


def _prows(w):
    idx = np.concatenate([_ILV + H * m for m in range(w.shape[0] // H)])
    return w[idx, :]


def _pvec(v):
    return v[_ILV]


def _sc_gat_body(xl_hbm, xr_hbm, e_hbm, src_hbm, dst_hbm, att_hbm,
                 out_hbm, den_hbm,
                 srcb0, dstb0, eb0, xlb0, xrb0, exb0,
                 srcb1, dstb1, eb1, xlb1, xrb1, exb1,
                 attb, zb, zd, lsem0, gsem0, lsem1, gsem1,
                 out_sh, den_sh):
    slot0 = (srcb0, dstb0, eb0, xlb0, xrb0, exb0, lsem0, gsem0)
    slot1 = (srcb1, dstb1, eb1, xlb1, xrb1, exb1, lsem1, gsem1)
    c = lax.axis_index("c")
    s = lax.axis_index("s")
    wid = s * 2 + c

    def _zrow(i, _):
        for j in range(H // 16):
            zb[i, pl.ds(j * 16, 16)] = jnp.zeros((16,), jnp.float32)
        return 0
    lax.fori_loop(0, 16, _zrow, 0)

    def _zd(i, _):
        zd[pl.ds(i * 16, 16)] = jnp.zeros((16,), jnp.float32)
        return 0
    lax.fori_loop(0, 40, _zd, 0)

    def _zcp(i, _):
        pltpu.sync_copy(zb, out_sh.at[pl.ds(s * 640 + i * 16, 16)])
        return 0
    lax.fori_loop(0, 40, _zcp, 0)
    pltpu.sync_copy(zd, den_sh.at[pl.ds(s * 640, 640)])

    pltpu.sync_copy(att_hbm, attb)
    att06 = [attb[pl.ds(16 * j, 16)] * 0.6 for j in range(H // 16)]
    att04 = [attb[pl.ds(16 * j, 16)] * 0.4 for j in range(H // 16)]

    plsc.subcore_barrier()

    riota = lax.broadcasted_iota(jnp.int32, (16,), 0)

    def _fire_linear(i, slot):
        srcb, dstb, eb, xlb, xrb, exb, lsem, gsem = slot
        base = wid * EDGES_PER_WORKER + i * CHUNK
        pltpu.async_copy(src_hbm.at[pl.ds(base, CHUNK)], srcb, lsem)
        pltpu.async_copy(dst_hbm.at[pl.ds(base, CHUNK)], dstb, lsem)
        pltpu.async_copy(e_hbm.at[pl.ds(base, CHUNK)], eb, lsem)

    def _wait_linear(slot):
        srcb, dstb, eb, xlb, xrb, exb, lsem, gsem = slot
        pltpu.make_async_copy(src_hbm.at[pl.ds(0, CHUNK)], srcb, lsem).wait()
        pltpu.make_async_copy(dst_hbm.at[pl.ds(0, CHUNK)], dstb, lsem).wait()
        pltpu.make_async_copy(e_hbm.at[pl.ds(0, CHUNK)], eb, lsem).wait()

    def _fire_gather(slot):
        srcb, dstb, eb, xlb, xrb, exb, lsem, gsem = slot
        pltpu.async_copy(xl_hbm.at[srcb], xlb, gsem)
        pltpu.async_copy(xr_hbm.at[dstb], xrb, gsem)

    def _wait_gather(slot):
        srcb, dstb, eb, xlb, xrb, exb, lsem, gsem = slot
        pltpu.make_async_copy(xl_hbm.at[srcb], xlb, gsem).wait()
        pltpu.make_async_copy(xr_hbm.at[dstb], xrb, gsem).wait()

    def _compute(slot):
        srcb, dstb, eb, xlb, xrb, exb, lsem, gsem = slot

        def _logit(i):
            prods = []
            for j in range(H // 16):
                sl = pl.ds(16 * j, 16)
                v = xlb[i, sl] + xrb[i, sl] + eb[i, sl]
                prods.append(att06[j] * v + att04[j] * jnp.abs(v))
            while len(prods) > 1:
                prods = [a + b for a, b in zip(prods[::2], prods[1::2])]
            acc = prods[0]
            for sh in (8, 4, 2, 1):
                acc = acc + _dg(acc, riota ^ sh)
            return acc

        for eg, npair in GROUPS:
            def _edge2(ii, tot, eg=eg):
                i = eg + 2 * ii
                a0 = _logit(i)
                a1 = _logit(i + 1)
                tot = jnp.where(riota == 2 * ii, a0, tot)
                return jnp.where(riota == 2 * ii + 1, a1, tot)

            tot = lax.fori_loop(0, npair, _edge2,
                                jnp.zeros((16,), jnp.float32))
            exv = jnp.exp(tot)
            exb[pl.ds(eg, 16)] = exv

            def _wt(ii, _, eg=eg, exv=exv):
                i = eg + 2 * ii
                bv0 = _dg(exv, riota * 0 + 2 * ii)
                bv1 = _dg(exv, riota * 0 + 2 * ii + 1)
                for j in range(H // 16):
                    sl = pl.ds(j * 16, 16)
                    xlb[i, sl] = xlb[i, sl] * bv0
                    xlb[i + 1, sl] = xlb[i + 1, sl] * bv1
                return 0

            lax.fori_loop(0, npair, _wt, 0)

    def _scatter(slot):
        srcb, dstb, eb, xlb, xrb, exb, lsem, gsem = slot
        pltpu.sync_copy(exb.at[pl.ds(0, CHUNK)], den_sh.at[dstb], add=True)
        pltpu.sync_copy(xlb, out_sh.at[dstb], add=True)

    def _half(i, cur, nxt):
        _fire_linear(i + 1, nxt)
        _wait_gather(cur)
        _wait_linear(nxt)
        _fire_gather(nxt)
        _compute(cur)
        _scatter(cur)

    _fire_linear(0, slot0)
    _wait_linear(slot0)
    _fire_gather(slot0)

    def _pair(pi, _):
        i = 2 * pi
        _half(i, slot0, slot1)
        _half(i + 1, slot1, slot0)
        return 0
    lax.fori_loop(0, (NCHUNK - 2) // 2, _pair, 0)

    _half(NCHUNK - 2, slot0, slot1)
    _wait_gather(slot1)
    _compute(slot1)
    _scatter(slot1)

    plsc.subcore_barrier()

    @pl.when(s == 0)
    def _copy_out():
        pltpu.sync_copy(out_sh, out_hbm.at[c])
        pltpu.sync_copy(den_sh, den_hbm.at[c])


def _pack32(x):
    return lax.bitcast_convert_type(
        x.reshape(x.shape[:-1] + (H // 2, 2)), jnp.int32)


def _sc_gat_layer(xl, xr, e, src, dst, att):
    mesh = plsc.VectorSubcoreMesh(core_axis_name="c", subcore_axis_name="s")

    f = pl.kernel(
        _sc_gat_body,
        out_type=[
            jax.ShapeDtypeStruct((2, NPAD, H), jnp.float32),
            jax.ShapeDtypeStruct((2, NPAD), jnp.float32),
        ],
        mesh=mesh,
        scratch_types=(
            [
                pltpu.VMEM((CHUNK,), jnp.int32),
                pltpu.VMEM((CHUNK,), jnp.int32),
                pltpu.VMEM((CHUNK, H), jnp.float32),
                pltpu.VMEM((CHUNK, H), jnp.float32),
                pltpu.VMEM((CHUNK, H), jnp.float32),
                pltpu.VMEM((CHUNK + 8,), jnp.float32),
            ] * 2
            + [
                pltpu.VMEM((H,), jnp.float32),
                pltpu.VMEM((16, H), jnp.float32),
                pltpu.VMEM((640,), jnp.float32),
                pltpu.SemaphoreType.DMA,
                pltpu.SemaphoreType.DMA,
                pltpu.SemaphoreType.DMA,
                pltpu.SemaphoreType.DMA,
                pltpu.VMEM_SHARED((NPAD, H), jnp.float32),
                pltpu.VMEM_SHARED((NPAD,), jnp.float32),
            ]
        ),
    )
    return f(xl, xr, e, src, dst, att)


def _e_proj_body(ea_ref, w_ref, o1, o2, o3, o4):
    ea = ea_ref[...]
    w = w_ref[...]
    for i, o in enumerate((o1, o2, o3, o4)):
        o[...] = jnp.dot(ea, w[i], preferred_element_type=jnp.float32)


def _e_proj(edge_attr, w_stack):
    BE = 4000
    grid = (E // BE,)
    return pl.pallas_call(
        _e_proj_body,
        grid=grid,
        in_specs=[
            pl.BlockSpec((BE, ED), lambda i: (i, 0)),
            pl.BlockSpec((4, ED, H), lambda i: (0, 0, 0)),
        ],
        out_specs=[pl.BlockSpec((BE, H), lambda i: (i, 0))] * 4,
        out_shape=[jax.ShapeDtypeStruct((E, H), jnp.float32)] * 4,
    )(edge_attr, w_stack)


def _lin1_body(x_ref, wl_ref, bl_ref, wr_ref, br_ref, xl_ref, xr_ref):
    x = x_ref[...]
    xl_ref[...] = jnp.dot(x, wl_ref[...], preferred_element_type=jnp.float32) + bl_ref[...]
    xr_ref[...] = jnp.dot(x, wr_ref[...], preferred_element_type=jnp.float32) + br_ref[...]


def _lin1(x, wl, bl, wr, br):
    return pl.pallas_call(
        _lin1_body,
        out_shape=[jax.ShapeDtypeStruct((N, H), jnp.float32)] * 2,
    )(x, wl, bl.reshape(1, H), wr, br.reshape(1, H))


def _post_gat(acc_ref, den_ref, bias_ref, g_ref, be_ref, a_ref):
    acc = acc_ref[0, :N] + acc_ref[1, :N]
    den = den_ref[0, :N] + den_ref[1, :N]
    h = acc / (den + 1e-16) + bias_ref[...]
    mu = jnp.mean(h, axis=0, keepdims=True)
    var = jnp.mean((h - mu) ** 2, axis=0, keepdims=True)
    hn = g_ref[...] * (h - mu) / jnp.sqrt(var + 1e-5) + be_ref[...]
    a = a_ref[0, 0]
    return jnp.where(hn >= 0, hn, a * hn)


def _mid_body(nprev, refs):
    (acc_ref, den_ref, bias_ref, g_ref, be_ref, a_ref) = refs[:6]
    prev = refs[6:6 + nprev]
    wl_ref, bl_ref, wr_ref, br_ref = refs[6 + nprev:6 + nprev + 4]
    h_ref, xl_ref, xr_ref = refs[6 + nprev + 4:]
    hv = _post_gat(acc_ref, den_ref, bias_ref, g_ref, be_ref, a_ref)
    h_ref[...] = hv
    inp = jnp.concatenate([p[...] for p in prev] + [hv], axis=1)
    xl_ref[...] = jnp.dot(inp, wl_ref[...], preferred_element_type=jnp.float32) + bl_ref[...]
    xr_ref[...] = jnp.dot(inp, wr_ref[...], preferred_element_type=jnp.float32) + br_ref[...]


def _mid(acc, den, bias, g, be, a, prev_hs, wl, bl, wr, br):
    nprev = len(prev_hs)

    def body(*refs):
        _mid_body(nprev, refs)

    return pl.pallas_call(
        body,
        out_shape=[jax.ShapeDtypeStruct((N, H), jnp.float32)] * 3,
    )(acc, den.reshape(2, NPAD, 1), bias.reshape(1, H), g.reshape(1, H),
      be.reshape(1, H), a.reshape(1, 1), *prev_hs,
      wl, bl.reshape(1, H), wr, br.reshape(1, H))


def _final_body(acc_ref, den_ref, bias_ref, h1_ref, h2_ref, h3_ref, b_ref,
                wm1_ref, bm1_ref, gm_ref, bem_ref, am_ref, wm2_ref, bm2_ref,
                lg_ref, pr_ref, pd_ref):
    acc = acc_ref[0, :N] + acc_ref[1, :N]
    den = den_ref[0, :N] + den_ref[1, :N]
    h4 = acc / (den + 1e-16) + bias_ref[...]
    hcat = jnp.concatenate([h1_ref[...], h2_ref[...], h3_ref[...], h4], axis=1)
    giota = lax.broadcasted_iota(jnp.int32, (1, G), 1)
    oh = (b_ref[...] == giota).astype(jnp.float32)
    pooled = lax.dot_general(oh, hcat, (((0,), (0,)), ((), ())),
                             preferred_element_type=jnp.float32)
    h = jnp.dot(pooled, wm1_ref[...], preferred_element_type=jnp.float32) + bm1_ref[...]
    mu = jnp.mean(h, axis=0, keepdims=True)
    var = jnp.mean((h - mu) ** 2, axis=0, keepdims=True)
    hn = gm_ref[...] * (h - mu) / jnp.sqrt(var + 1e-5) + bem_ref[...]
    am = am_ref[0, 0]
    h = jnp.where(hn >= 0, hn, am * hn)
    logits = jnp.dot(h, wm2_ref[...], preferred_element_type=jnp.float32) + bm2_ref[...]
    lg_ref[...] = logits
    ex = jnp.exp(logits - jnp.max(logits, axis=1, keepdims=True))
    prob = ex / jnp.sum(ex, axis=1, keepdims=True)
    pr_ref[...] = prob
    pmax = jnp.max(prob, axis=1, keepdims=True)
    cidx = lax.broadcasted_iota(jnp.int32, (G, OUT), 1)
    pd_ref[...] = jnp.min(jnp.where(prob == pmax, cidx, OUT), axis=1, keepdims=True)


def _final(acc, den, bias, h1, h2, h3, batch, wm1, bm1, gm, bem, am, wm2, bm2):
    return pl.pallas_call(
        _final_body,
        out_shape=[
            jax.ShapeDtypeStruct((G, OUT), jnp.float32),
            jax.ShapeDtypeStruct((G, OUT), jnp.float32),
            jax.ShapeDtypeStruct((G, 1), jnp.int32),
        ],
    )(acc, den.reshape(2, NPAD, 1), bias.reshape(1, H), h1, h2, h3,
      batch.reshape(N, 1), wm1, bm1.reshape(1, H), gm.reshape(1, H),
      bem.reshape(1, H), am.reshape(1, 1), wm2, bm2.reshape(1, OUT))


def kernel(x, edge_index, edge_attr, batch, params):
    p = params
    src = edge_index[0]
    dst = edge_index[1]

    w_stack = jnp.stack([_pcols(p['We1']), _pcols(p['We2']),
                         _pcols(p['We3']), _pcols(p['We4'])])
    e1, e2, e3, e4 = _e_proj(edge_attr, w_stack)

    xl, xr = _lin1(x, _pcols(p['Wl1']), _pvec(p['bl1']),
                   _pcols(p['Wr1']), _pvec(p['br1']))
    acc1, den1 = _sc_gat_layer(xl, xr, e1, src, dst, _pvec(p['att1']))

    h1, xl, xr = _mid(acc1, den1, _pvec(p['bias1']), _pvec(p['g1']),
                      _pvec(p['be1']), p['a1'],
                      [], _pcols(_prows(p['Wl2'])), _pvec(p['bl2']),
                      _pcols(_prows(p['Wr2'])), _pvec(p['br2']))
    acc2, den2 = _sc_gat_layer(xl, xr, e2, src, dst, _pvec(p['att2']))

    h2, xl, xr = _mid(acc2, den2, _pvec(p['bias2']), _pvec(p['g2']),
                      _pvec(p['be2']), p['a2'],
                      [h1], _pcols(_prows(p['Wl3'])), _pvec(p['bl3']),
                      _pcols(_prows(p['Wr3'])), _pvec(p['br3']))
    acc3, den3 = _sc_gat_layer(xl, xr, e3, src, dst, _pvec(p['att3']))

    h3, xl, xr = _mid(acc3, den3, _pvec(p['bias3']), _pvec(p['g3']),
                      _pvec(p['be3']), p['a3'],
                      [h1, h2], _pcols(_prows(p['Wl4'])), _pvec(p['bl4']),
                      _pcols(_prows(p['Wr4'])), _pvec(p['br4']))
    acc4, den4 = _sc_gat_layer(xl, xr, e4, src, dst, _pvec(p['att4']))

    logits, prob, pred = _final(acc4, den4, _pvec(p['bias4']), h1, h2, h3,
                                batch, _prows(p['Wm1']), p['bm1'], p['gm'],
                                p['bem'], p['am'], p['Wm2'], p['bm2'])
    return logits, prob, pred.reshape(G)
